# baseline (speedup 1.0000x reference)
import functools

import jax
import jax.numpy as jnp
from jax import lax
from jax.experimental import pallas as pl
from jax.experimental.pallas import tpu as pltpu
from jax.experimental.pallas import tpu_sc as plsc

N = 10000
E = 320000
HID = 256
F_OUT = 128

CHUNK = 64
EPAD = 327680
NCH = EPAD // CHUNK
NDEG = 10240
NACC = 10112
NTILE = 16
LANES = 16
NBUF = 5
GAHEAD = 3
G = 8
RB = 2000

_vmesh = plsc.VectorSubcoreMesh(core_axis_name="c", subcore_axis_name="s")


@functools.partial(
    pl.kernel,
    out_type=jax.ShapeDtypeStruct((32 * NDEG,), jnp.float32),
    mesh=_vmesh,
    compiler_params=pltpu.CompilerParams(needs_layout_passes=False),
    scratch_types=[
        pltpu.VMEM((NDEG,), jnp.float32),
        pltpu.VMEM((NCH // 32, CHUNK), jnp.int32),
    ],
)
def _deg_kernel(dst_hbm, zdeg_hbm, out_hbm, dloc, didx):
    c = lax.axis_index("c")
    s = lax.axis_index("s")
    wid = c * NTILE + s
    nch = NCH // 32
    pltpu.sync_copy(zdeg_hbm, dloc)
    pltpu.sync_copy(dst_hbm.at[pl.ds(wid * nch, nch)], didx)
    ones = jnp.full((LANES,), 1.0, jnp.float32)

    @pl.loop(0, nch)
    def _(j):
        for k in range(CHUNK // LANES):
            idx = didx[j, pl.ds(k * LANES, LANES)]
            plsc.addupdate_scatter(dloc, [idx], ones)

    pltpu.sync_copy(dloc, out_hbm.at[pl.ds(wid * NDEG, NDEG)])


def _make_agg(feature_split):
    nch = NCH // NTILE if feature_split else NCH // 32
    rz = NACC // NTILE

    @functools.partial(
        pl.kernel,
        out_type=jax.ShapeDtypeStruct((2 * NACC, 128), jnp.float32),
        mesh=_vmesh,
        scratch_types=(
            [pltpu.VMEM((G, CHUNK), jnp.int32),
             pltpu.VMEM((G, CHUNK), jnp.int32)]
            + [pltpu.VMEM((CHUNK, 128), jnp.float32) for _ in range(NBUF)]
            + [pltpu.VMEM_SHARED((NACC, 128), jnp.float32)]
            + [pltpu.SemaphoreType.DMA for _ in range(2 * NBUF)]
        ),
    )
    def agg(x_hbm, src_hbm, dst_hbm, zrows_hbm, out_hbm, sidx, didx, *rest):
        bufs = rest[:NBUF]
        acc = rest[NBUF]
        gsems = rest[NBUF + 1:2 * NBUF + 1]
        ssems = rest[2 * NBUF + 1:]
        c = lax.axis_index("c")
        s = lax.axis_index("s")
        pltpu.sync_copy(zrows_hbm, acc.at[pl.ds(s * rz, rz)])
        base = s * nch if feature_split else (c * NTILE + s) * nch
        off = c * NACC
        plsc.subcore_barrier()

        @pl.loop(0, nch // G)
        def _(g):
            pltpu.sync_copy(src_hbm.at[pl.ds(base + g * G, G)], sidx)
            pltpu.sync_copy(dst_hbm.at[pl.ds(base + g * G, G)], didx)
            if feature_split:
                for j in range(G):
                    for k in range(CHUNK // LANES):
                        sl = (j, pl.ds(k * LANES, LANES))
                        sidx[sl] = sidx[sl] + off
            slag = NBUF - GAHEAD
            gd = [None] * G
            sd = [None] * G
            for j in range(min(GAHEAD, G)):
                gd[j] = pltpu.async_copy(
                    x_hbm.at[sidx.at[j]], bufs[j % NBUF], gsems[j % NBUF])
            for j in range(G):
                b = j % NBUF
                if j >= slag and sd[j - slag] is not None:
                    sd[j - slag].wait()
                gd[j].wait()
                if j + GAHEAD < G:
                    nb = (j + GAHEAD) % NBUF
                    gd[j + GAHEAD] = pltpu.async_copy(
                        x_hbm.at[sidx.at[j + GAHEAD]], bufs[nb], gsems[nb])
                sd[j] = pltpu.async_copy(
                    bufs[b], acc.at[didx.at[j]], ssems[b], add=True)
            for j in range(max(0, G - slag), G):
                if sd[j] is not None:
                    sd[j].wait()

        plsc.subcore_barrier()
        pltpu.sync_copy(acc.at[pl.ds(s * rz, rz)],
                        out_hbm.at[pl.ds(c * NACC + s * rz, rz)])

    return agg


_agg_edge = _make_agg(False)
_agg_feat = _make_agg(True)


def _dinv_col(degp_ref):
    deg = jnp.sum(degp_ref[...], axis=1, keepdims=True) + 1.0
    return lax.rsqrt(deg)


def _mm(a, b):
    return lax.dot_general(a, b, (((1,), (0,)), ((), ())),
                           preferred_element_type=jnp.float32,
                           precision=lax.Precision.HIGHEST)


def _a_body(degp_ref, x_ref, xs_ref):
    dcol = _dinv_col(degp_ref)
    xs_ref[...] = dcol * x_ref[...]


def _b_body(degp_ref, s1_ref, x_ref, w1_ref, b1_ref, h1_ref, ha_ref, hb_ref):
    dcol = _dinv_col(degp_ref)
    z1 = dcol * (s1_ref[0] + s1_ref[1]) + (dcol * dcol) * x_ref[...]
    h1 = jnp.maximum(_mm(z1, w1_ref[...]) + b1_ref[...], 0.0)
    h1_ref[...] = h1
    ha_ref[...] = dcol * h1[:, :128]
    hb_ref[...] = dcol * h1[:, 128:]


def _c_body(degp_ref, s2_ref, h1_ref, w2_ref, w3_ref, b2_ref, t_ref, ts_ref):
    dcol = _dinv_col(degp_ref)
    h1 = h1_ref[...]
    d2 = dcol * dcol
    z2 = jnp.concatenate(
        [dcol * s2_ref[0] + d2 * h1[:, :128],
         dcol * s2_ref[1] + d2 * h1[:, 128:]], axis=1)
    h2 = jnp.maximum(_mm(z2, w2_ref[...]) + b2_ref[...], 0.0)
    t = _mm(h2, w3_ref[...])
    t_ref[...] = t
    ts_ref[...] = dcol * t


def _d_body(degp_ref, s3_ref, t_ref, b3_ref, o_ref):
    dcol = _dinv_col(degp_ref)
    o_ref[...] = (dcol * (s3_ref[0] + s3_ref[1])
                  + (dcol * dcol) * t_ref[...] + b3_ref[...])


def _rowspec(w):
    return pl.BlockSpec((RB, w), lambda i: (i, 0))


def _pairspec():
    return pl.BlockSpec((2, RB, 128), lambda i: (0, i, 0))


def _fullspec(shape):
    return pl.BlockSpec(shape, lambda i: tuple(0 for _ in shape))


_stage_a = pl.pallas_call(
    _a_body, grid=(N // RB,),
    in_specs=[_rowspec(32), _rowspec(128)],
    out_specs=_rowspec(128),
    out_shape=jax.ShapeDtypeStruct((N, 128), jnp.float32))

_stage_b = pl.pallas_call(
    _b_body, grid=(N // RB,),
    in_specs=[_rowspec(32), _pairspec(), _rowspec(128),
              _fullspec((128, 256)), _fullspec((1, 256))],
    out_specs=[_rowspec(256), _rowspec(128), _rowspec(128)],
    out_shape=[jax.ShapeDtypeStruct((N, 256), jnp.float32),
               jax.ShapeDtypeStruct((N, 128), jnp.float32),
               jax.ShapeDtypeStruct((N, 128), jnp.float32)])

_stage_c = pl.pallas_call(
    _c_body, grid=(N // RB,),
    in_specs=[_rowspec(32), _pairspec(), _rowspec(256),
              _fullspec((256, 256)), _fullspec((256, 128)),
              _fullspec((1, 256))],
    out_specs=[_rowspec(128), _rowspec(128)],
    out_shape=[jax.ShapeDtypeStruct((N, 128), jnp.float32),
               jax.ShapeDtypeStruct((N, 128), jnp.float32)])

_stage_d = pl.pallas_call(
    _d_body, grid=(N // RB,),
    in_specs=[_rowspec(32), _pairspec(), _rowspec(128),
              _fullspec((1, 128))],
    out_specs=_rowspec(128),
    out_shape=jax.ShapeDtypeStruct((N, 128), jnp.float32))


def kernel(x, edge_index, W1, b1, W2, b2, W3, b3):
    src = edge_index[0].astype(jnp.int32)
    dst = edge_index[1].astype(jnp.int32)
    pad = EPAD - E
    srcp = jnp.concatenate([src, jnp.zeros((pad,), jnp.int32)]).reshape(NCH, CHUNK)
    dstp = jnp.concatenate([dst, jnp.full((pad,), N, jnp.int32)]).reshape(NCH, CHUNK)
    zdeg = jnp.zeros((NDEG,), jnp.float32)
    zrows = jnp.zeros((NACC // NTILE, 128), jnp.float32)

    degp = _deg_kernel(dstp, zdeg)
    degpt = degp.reshape(32, NDEG).T

    zpad = jnp.zeros((NACC - N, 128), jnp.float32)
    xs = _stage_a(degpt, x)
    s1 = _agg_edge(jnp.concatenate([xs, zpad]), srcp, dstp,
                   zrows).reshape(2, NACC, 128)
    h1, ha, hb = _stage_b(degpt, s1, x, W1, b1.reshape(1, HID))
    hcat = jnp.concatenate([ha, zpad, hb, zpad], axis=0)
    s2 = _agg_feat(hcat, srcp, dstp, zrows).reshape(2, NACC, 128)
    t, ts = _stage_c(degpt, s2, h1, W2, W3, b2.reshape(1, HID))
    s3 = _agg_edge(jnp.concatenate([ts, zpad]), srcp, dstp,
                   zrows).reshape(2, NACC, 128)
    out = _stage_d(degpt, s3, t, b3.reshape(1, F_OUT))
    return out


# speedup vs baseline: 1.2045x; 1.2045x over previous
import functools

import jax
import jax.numpy as jnp
from jax import lax
from jax.experimental import pallas as pl
from jax.experimental.pallas import tpu as pltpu
from jax.experimental.pallas import tpu_sc as plsc

N = 10000
E = 320000
HID = 256
F_OUT = 128

CHUNK = 64
EPAD = 327680
NCH = EPAD // CHUNK
NDEG = 10240
NACC = 10112
NTILE = 16
LANES = 16
NG = 3
NS = 3
GAHEAD = 2
SLAG = 2
G = 8
RB = 2000

import numpy as _np
_PERM = _np.empty((128,), _np.int32)
for _k in range(4):
    for _i in range(16):
        _PERM[32 * _k + 2 * _i] = 32 * _k + _i
        _PERM[32 * _k + 2 * _i + 1] = 32 * _k + 16 + _i

_vmesh = plsc.VectorSubcoreMesh(core_axis_name="c", subcore_axis_name="s")


@functools.partial(
    pl.kernel,
    out_type=jax.ShapeDtypeStruct((32 * NDEG,), jnp.float32),
    mesh=_vmesh,
    compiler_params=pltpu.CompilerParams(needs_layout_passes=False),
    scratch_types=[
        pltpu.VMEM((NDEG,), jnp.float32),
        pltpu.VMEM((NCH // 32, CHUNK), jnp.int32),
    ],
)
def _deg_kernel(dst_hbm, zdeg_hbm, out_hbm, dloc, didx):
    c = lax.axis_index("c")
    s = lax.axis_index("s")
    wid = c * NTILE + s
    nch = NCH // 32
    pltpu.sync_copy(zdeg_hbm, dloc)
    pltpu.sync_copy(dst_hbm.at[pl.ds(wid * nch, nch)], didx)
    ones = jnp.full((LANES,), 1.0, jnp.float32)

    @pl.loop(0, nch)
    def _(j):
        for k in range(CHUNK // LANES):
            idx = didx[j, pl.ds(k * LANES, LANES)]
            plsc.addupdate_scatter(dloc, [idx], ones)

    pltpu.sync_copy(dloc, out_hbm.at[pl.ds(wid * NDEG, NDEG)])


def _make_agg(feature_split):
    nch = NCH // NTILE if feature_split else NCH // 32
    rz = NACC // NTILE

    @functools.partial(
        pl.kernel,
        out_type=jax.ShapeDtypeStruct((2 * NACC, 128), jnp.float32),
        mesh=_vmesh,
        compiler_params=pltpu.CompilerParams(needs_layout_passes=False,
                                             use_tc_tiling_on_sc=False),
        scratch_types=(
            [pltpu.VMEM((G, CHUNK), jnp.int32),
             pltpu.VMEM((G, CHUNK), jnp.int32)]
            + [pltpu.VMEM((CHUNK, 64), jnp.int32) for _ in range(NG)]
            + [pltpu.VMEM((CHUNK, 128), jnp.float32) for _ in range(NS)]
            + [pltpu.VMEM_SHARED((NACC, 128), jnp.float32)]
            + [pltpu.SemaphoreType.DMA for _ in range(NG + NS)]
        ),
    )
    def agg(x_hbm, src_hbm, dst_hbm, zrows_hbm, out_hbm, sidx, didx, *rest):
        gbufs = rest[:NG]
        fbufs = rest[NG:NG + NS]
        acc = rest[NG + NS]
        gsems = rest[NG + NS + 1:NG + NS + 1 + NG]
        ssems = rest[NG + NS + 1 + NG:]
        c = lax.axis_index("c")
        s = lax.axis_index("s")
        pltpu.sync_copy(zrows_hbm, acc.at[pl.ds(s * rz, rz)])
        base = s * nch if feature_split else (c * NTILE + s) * nch
        off = c * NACC
        plsc.subcore_barrier()

        @pl.loop(0, nch // G)
        def _(g):
            pltpu.sync_copy(src_hbm.at[pl.ds(base + g * G, G)], sidx)
            pltpu.sync_copy(dst_hbm.at[pl.ds(base + g * G, G)], didx)
            if feature_split:
                for j in range(G):
                    for k in range(CHUNK // LANES):
                        sl = (j, pl.ds(k * LANES, LANES))
                        sidx[sl] = sidx[sl] + off
            gd = [None] * G
            sd = [None] * G
            for j in range(min(GAHEAD, G)):
                gd[j] = pltpu.async_copy(
                    x_hbm.at[sidx.at[j]], gbufs[j % NG], gsems[j % NG])
            for j in range(G):
                if j >= SLAG and sd[j - SLAG] is not None:
                    sd[j - SLAG].wait()
                gd[j].wait()
                if j + GAHEAD < G:
                    nb = (j + GAHEAD) % NG
                    gd[j + GAHEAD] = pltpu.async_copy(
                        x_hbm.at[sidx.at[j + GAHEAD]], gbufs[nb], gsems[nb])
                bsrc = gbufs[j % NG]
                fdst = fbufs[j % NS]

                @pl.loop(0, CHUNK)
                def _(r):
                    for k in range(4):
                        w = bsrc[r, pl.ds(16 * k, 16)]
                        lo = plsc.bitcast(w << 16, jnp.float32)
                        hi = plsc.bitcast(w & jnp.int32(-65536), jnp.float32)
                        fdst[r, pl.ds(32 * k, 16)] = lo
                        fdst[r, pl.ds(32 * k + 16, 16)] = hi

                sd[j] = pltpu.async_copy(
                    fdst, acc.at[didx.at[j]], ssems[j % NS], add=True)
            for j in range(max(0, G - SLAG), G):
                if sd[j] is not None:
                    sd[j].wait()

        plsc.subcore_barrier()
        pltpu.sync_copy(acc.at[pl.ds(s * rz, rz)],
                        out_hbm.at[pl.ds(c * NACC + s * rz, rz)])

    return agg


_agg_edge = _make_agg(False)
_agg_feat = _make_agg(True)


def _dinv_col(degp_ref):
    deg = jnp.sum(degp_ref[...], axis=1, keepdims=True) + 1.0
    return lax.rsqrt(deg)


def _mm(a, b):
    return lax.dot_general(a, b, (((1,), (0,)), ((), ())),
                           preferred_element_type=jnp.float32,
                           precision=lax.Precision.HIGHEST)


def _a_body(degp_ref, x_ref, xs_ref):
    dcol = _dinv_col(degp_ref)
    xs_ref[...] = dcol * x_ref[...]


def _b_body(degp_ref, s1_ref, x_ref, w1_ref, b1_ref, h1_ref, ha_ref, hb_ref):
    dcol = _dinv_col(degp_ref)
    z1 = dcol * (s1_ref[0] + s1_ref[1]) + (dcol * dcol) * x_ref[...]
    h1 = jnp.maximum(_mm(z1, w1_ref[...]) + b1_ref[...], 0.0)
    h1_ref[...] = h1
    ha_ref[...] = dcol * h1[:, :128]
    hb_ref[...] = dcol * h1[:, 128:]


def _c_body(degp_ref, s2_ref, h1_ref, w2_ref, w3_ref, b2_ref, t_ref, ts_ref):
    dcol = _dinv_col(degp_ref)
    h1 = h1_ref[...]
    d2 = dcol * dcol
    z2 = jnp.concatenate(
        [dcol * s2_ref[0] + d2 * h1[:, :128],
         dcol * s2_ref[1] + d2 * h1[:, 128:]], axis=1)
    h2 = jnp.maximum(_mm(z2, w2_ref[...]) + b2_ref[...], 0.0)
    t = _mm(h2, w3_ref[...])
    t_ref[...] = t
    ts_ref[...] = dcol * t


def _d_body(degp_ref, s3_ref, t_ref, b3_ref, o_ref):
    dcol = _dinv_col(degp_ref)
    o_ref[...] = (dcol * (s3_ref[0] + s3_ref[1])
                  + (dcol * dcol) * t_ref[...] + b3_ref[...])


def _rowspec(w):
    return pl.BlockSpec((RB, w), lambda i: (i, 0))


def _pairspec():
    return pl.BlockSpec((2, RB, 128), lambda i: (0, i, 0))


def _fullspec(shape):
    return pl.BlockSpec(shape, lambda i: tuple(0 for _ in shape))


_stage_a = pl.pallas_call(
    _a_body, grid=(N // RB,),
    in_specs=[_rowspec(32), _rowspec(128)],
    out_specs=_rowspec(128),
    out_shape=jax.ShapeDtypeStruct((N, 128), jnp.float32))

_stage_b = pl.pallas_call(
    _b_body, grid=(N // RB,),
    in_specs=[_rowspec(32), _pairspec(), _rowspec(128),
              _fullspec((128, 256)), _fullspec((1, 256))],
    out_specs=[_rowspec(256), _rowspec(128), _rowspec(128)],
    out_shape=[jax.ShapeDtypeStruct((N, 256), jnp.float32),
               jax.ShapeDtypeStruct((N, 128), jnp.float32),
               jax.ShapeDtypeStruct((N, 128), jnp.float32)])

_stage_c = pl.pallas_call(
    _c_body, grid=(N // RB,),
    in_specs=[_rowspec(32), _pairspec(), _rowspec(256),
              _fullspec((256, 256)), _fullspec((256, 128)),
              _fullspec((1, 256))],
    out_specs=[_rowspec(128), _rowspec(128)],
    out_shape=[jax.ShapeDtypeStruct((N, 128), jnp.float32),
               jax.ShapeDtypeStruct((N, 128), jnp.float32)])

_stage_d = pl.pallas_call(
    _d_body, grid=(N // RB,),
    in_specs=[_rowspec(32), _pairspec(), _rowspec(128),
              _fullspec((1, 128))],
    out_specs=_rowspec(128),
    out_shape=jax.ShapeDtypeStruct((N, 128), jnp.float32))


def kernel(x, edge_index, W1, b1, W2, b2, W3, b3):
    src = edge_index[0].astype(jnp.int32)
    dst = edge_index[1].astype(jnp.int32)
    pad = EPAD - E
    srcp = jnp.concatenate([src, jnp.zeros((pad,), jnp.int32)]).reshape(NCH, CHUNK)
    dstp = jnp.concatenate([dst, jnp.full((pad,), N, jnp.int32)]).reshape(NCH, CHUNK)
    zdeg = jnp.zeros((NDEG,), jnp.float32)
    zrows = jnp.zeros((NACC // NTILE, 128), jnp.float32)

    degp = _deg_kernel(dstp, zdeg)
    degpt = degp.reshape(32, NDEG).T

    zpad = jnp.zeros((NACC - N, 64), jnp.int32)

    def to_bf(a):
        ab = a.astype(jnp.bfloat16)[:, _PERM]
        return jax.lax.bitcast_convert_type(
            ab.reshape(a.shape[0], 64, 2), jnp.int32)

    xs = _stage_a(degpt, x)
    s1 = _agg_edge(jnp.concatenate([to_bf(xs), zpad]), srcp, dstp,
                   zrows).reshape(2, NACC, 128)
    h1, ha, hb = _stage_b(degpt, s1, x, W1, b1.reshape(1, HID))
    hcat = jnp.concatenate([to_bf(ha), zpad, to_bf(hb), zpad], axis=0)
    s2 = _agg_feat(hcat, srcp, dstp, zrows).reshape(2, NACC, 128)
    t, ts = _stage_c(degpt, s2, h1, W2, W3, b2.reshape(1, HID))
    s3 = _agg_edge(jnp.concatenate([to_bf(ts), zpad]), srcp, dstp,
                   zrows).reshape(2, NACC, 128)
    out = _stage_d(degpt, s3, t, b3.reshape(1, F_OUT))
    return out


# speedup vs baseline: 1.8868x; 1.5665x over previous
import functools

import jax
import jax.numpy as jnp
from jax import lax
from jax.experimental import pallas as pl
from jax.experimental.pallas import tpu as pltpu
from jax.experimental.pallas import tpu_sc as plsc

N = 10000
E = 320000
HID = 256
F_OUT = 128

CHUNK = 64
EPAD = 327680
NCH = EPAD // CHUNK
NDEG = 10240
NACC = 10112
NTILE = 16
LANES = 16
NBUF = 5
GAHEAD = 2
G = 8
RB = 2000

_NCH_T = NCH // NTILE
_RZ = NACC // NTILE

_vmesh = plsc.VectorSubcoreMesh(core_axis_name="c", subcore_axis_name="s")


@functools.partial(
    pl.kernel,
    out_type=jax.ShapeDtypeStruct((32 * NDEG,), jnp.float32),
    mesh=_vmesh,
    compiler_params=pltpu.CompilerParams(needs_layout_passes=False),
    scratch_types=[
        pltpu.VMEM((NDEG,), jnp.float32),
        pltpu.VMEM((NCH // 32, CHUNK), jnp.int32),
    ],
)
def _deg_kernel(dst_hbm, zdeg_hbm, out_hbm, dloc, didx):
    c = lax.axis_index("c")
    s = lax.axis_index("s")
    wid = c * NTILE + s
    nch = NCH // 32
    pltpu.sync_copy(zdeg_hbm, dloc)
    pltpu.sync_copy(dst_hbm.at[pl.ds(wid * nch, nch)], didx)
    ones = jnp.full((LANES,), 1.0, jnp.float32)

    @pl.loop(0, nch)
    def _(j):
        for k in range(CHUNK // LANES):
            idx = didx[j, pl.ds(k * LANES, LANES)]
            plsc.addupdate_scatter(dloc, [idx], ones)

    pltpu.sync_copy(dloc, out_hbm.at[pl.ds(wid * NDEG, NDEG)])


@functools.partial(
    pl.kernel,
    out_type=jax.ShapeDtypeStruct((2 * NACC, 64), jnp.float32),
    mesh=_vmesh,
    compiler_params=pltpu.CompilerParams(needs_layout_passes=False,
                                         use_tc_tiling_on_sc=False),
    scratch_types=(
        [pltpu.VMEM((G, CHUNK), jnp.int32),
         pltpu.VMEM((G, CHUNK), jnp.int32)]
        + [pltpu.VMEM((CHUNK, 64), jnp.float32) for _ in range(NBUF)]
        + [pltpu.VMEM_SHARED((NACC, 64), jnp.float32),
           pltpu.VMEM_SHARED((NACC, 64), jnp.float32)]
        + [pltpu.SemaphoreType.DMA for _ in range(2 * NBUF)]
    ),
)
def _agg64(xpair_hbm, src_hbm, dst_hbm, zrows_hbm, out_hbm,
           sidx, didx, *rest):
    bufs = rest[:NBUF]
    xsp = rest[NBUF]
    acc = rest[NBUF + 1]
    gsems = rest[NBUF + 2:NBUF + 2 + NBUF]
    ssems = rest[NBUF + 2 + NBUF:]
    c = lax.axis_index("c")
    s = lax.axis_index("s")
    rsl = pl.ds(s * _RZ, _RZ)
    pltpu.sync_copy(xpair_hbm.at[pl.ds(c * NACC + s * _RZ, _RZ)],
                    xsp.at[rsl])
    pltpu.sync_copy(zrows_hbm, acc.at[rsl])
    base = s * _NCH_T
    plsc.subcore_barrier()

    @pl.loop(0, _NCH_T // G)
    def _(g):
        pltpu.sync_copy(src_hbm.at[pl.ds(base + g * G, G)], sidx)
        pltpu.sync_copy(dst_hbm.at[pl.ds(base + g * G, G)], didx)
        slag = NBUF - GAHEAD
        gd = [None] * G
        sd = [None] * G
        for j in range(min(GAHEAD, G)):
            gd[j] = pltpu.async_copy(
                xsp.at[sidx.at[j]], bufs[j % NBUF], gsems[j % NBUF])
        for j in range(G):
            b = j % NBUF
            if j >= slag and sd[j - slag] is not None:
                sd[j - slag].wait()
            gd[j].wait()
            if j + GAHEAD < G:
                nb = (j + GAHEAD) % NBUF
                gd[j + GAHEAD] = pltpu.async_copy(
                    xsp.at[sidx.at[j + GAHEAD]], bufs[nb], gsems[nb])
            sd[j] = pltpu.async_copy(
                bufs[b], acc.at[didx.at[j]], ssems[b], add=True)
        for j in range(max(0, G - slag), G):
            if sd[j] is not None:
                sd[j].wait()

    plsc.subcore_barrier()
    pltpu.sync_copy(acc.at[rsl], out_hbm.at[pl.ds(c * NACC + s * _RZ, _RZ)])


def _dinv_col(degp_ref):
    deg = jnp.sum(degp_ref[...], axis=1, keepdims=True) + 1.0
    return lax.rsqrt(deg)


def _mm(a, b):
    return lax.dot_general(a, b, (((1,), (0,)), ((), ())),
                           preferred_element_type=jnp.float32,
                           precision=lax.Precision.HIGHEST)


def _a_body(degp_ref, x_ref, xp_ref):
    dcol = _dinv_col(degp_ref)
    xsq = dcol * x_ref[...]
    xp_ref[0] = xsq[:, :64]
    xp_ref[1] = xsq[:, 64:]


def _b_body(degp_ref, s1_ref, x_ref, w1_ref, b1_ref,
            h1_ref, hpa_ref, hpb_ref):
    dcol = _dinv_col(degp_ref)
    ssum = jnp.concatenate([s1_ref[0], s1_ref[1]], axis=1)
    z1 = dcol * ssum + (dcol * dcol) * x_ref[...]
    h1 = jnp.maximum(_mm(z1, w1_ref[...]) + b1_ref[...], 0.0)
    h1_ref[...] = h1
    hs = dcol * h1
    hpa_ref[0] = hs[:, 0:64]
    hpa_ref[1] = hs[:, 64:128]
    hpb_ref[0] = hs[:, 128:192]
    hpb_ref[1] = hs[:, 192:256]


def _c_body(degp_ref, s2a_ref, s2b_ref, h1_ref, w2_ref, w3_ref, b2_ref,
            t_ref, tp_ref):
    dcol = _dinv_col(degp_ref)
    ssum = jnp.concatenate(
        [s2a_ref[0], s2a_ref[1], s2b_ref[0], s2b_ref[1]], axis=1)
    z2 = dcol * ssum + (dcol * dcol) * h1_ref[...]
    h2 = jnp.maximum(_mm(z2, w2_ref[...]) + b2_ref[...], 0.0)
    t = _mm(h2, w3_ref[...])
    t_ref[...] = t
    ts = dcol * t
    tp_ref[0] = ts[:, :64]
    tp_ref[1] = ts[:, 64:]


def _d_body(degp_ref, s3_ref, t_ref, b3_ref, o_ref):
    dcol = _dinv_col(degp_ref)
    ssum = jnp.concatenate([s3_ref[0], s3_ref[1]], axis=1)
    o_ref[...] = (dcol * ssum + (dcol * dcol) * t_ref[...] + b3_ref[...])


def _rowspec(w):
    return pl.BlockSpec((RB, w), lambda i: (i, 0))


def _pairspec():
    return pl.BlockSpec((2, RB, 64), lambda i: (0, i, 0))


def _fullspec(shape):
    return pl.BlockSpec(shape, lambda i: tuple(0 for _ in shape))


def _pair_out():
    return jax.ShapeDtypeStruct((2, NACC, 64), jnp.float32)


_stage_a = pl.pallas_call(
    _a_body, grid=(N // RB,),
    in_specs=[_rowspec(32), _rowspec(128)],
    out_specs=_pairspec(),
    out_shape=_pair_out())

_stage_b = pl.pallas_call(
    _b_body, grid=(N // RB,),
    in_specs=[_rowspec(32), _pairspec(), _rowspec(128),
              _fullspec((128, 256)), _fullspec((1, 256))],
    out_specs=[_rowspec(256), _pairspec(), _pairspec()],
    out_shape=[jax.ShapeDtypeStruct((N, 256), jnp.float32),
               _pair_out(), _pair_out()])

_stage_c = pl.pallas_call(
    _c_body, grid=(N // RB,),
    in_specs=[_rowspec(32), _pairspec(), _pairspec(), _rowspec(256),
              _fullspec((256, 256)), _fullspec((256, 128)),
              _fullspec((1, 256))],
    out_specs=[_rowspec(128), _pairspec()],
    out_shape=[jax.ShapeDtypeStruct((N, 128), jnp.float32), _pair_out()])

_stage_d = pl.pallas_call(
    _d_body, grid=(N // RB,),
    in_specs=[_rowspec(32), _pairspec(), _rowspec(128),
              _fullspec((1, 128))],
    out_specs=_rowspec(128),
    out_shape=jax.ShapeDtypeStruct((N, 128), jnp.float32))


def kernel(x, edge_index, W1, b1, W2, b2, W3, b3):
    src = edge_index[0].astype(jnp.int32)
    dst = edge_index[1].astype(jnp.int32)
    pad = EPAD - E
    srcp = jnp.concatenate([src, jnp.zeros((pad,), jnp.int32)]).reshape(NCH, CHUNK)
    dstp = jnp.concatenate([dst, jnp.full((pad,), N, jnp.int32)]).reshape(NCH, CHUNK)
    zdeg = jnp.zeros((NDEG,), jnp.float32)
    zrows = jnp.zeros((_RZ, 64), jnp.float32)

    def agg(pair):
        flat = _agg64(pair.reshape(2 * NACC, 64), srcp, dstp, zrows)
        return flat.reshape(2, NACC, 64)

    degp = _deg_kernel(dstp, zdeg)
    degpt = degp.reshape(32, NDEG).T

    xp = _stage_a(degpt, x)
    s1 = agg(xp)
    h1, hpa, hpb = _stage_b(degpt, s1, x, W1, b1.reshape(1, HID))
    s2a = agg(hpa)
    s2b = agg(hpb)
    t, tp = _stage_c(degpt, s2a, s2b, h1, W2, W3, b2.reshape(1, HID))
    s3 = agg(tp)
    out = _stage_d(degpt, s3, t, b3.reshape(1, F_OUT))
    return out


# speedup vs baseline: 2.1112x; 1.1189x over previous
import functools

import jax
import jax.numpy as jnp
from jax import lax
from jax.experimental import pallas as pl
from jax.experimental.pallas import tpu as pltpu
from jax.experimental.pallas import tpu_sc as plsc

N = 10000
E = 320000
HID = 256
F_OUT = 128

CHUNK = 128
EPAD = 327680
NCH = EPAD // CHUNK
NDEG = 10240
NACC = 10112
NTILE = 16
LANES = 16
NBUF = 5
GAHEAD = 2
G = 8
RB = 2000

_NCH_T = NCH // NTILE
_RZ = NACC // NTILE

_vmesh = plsc.VectorSubcoreMesh(core_axis_name="c", subcore_axis_name="s")


@functools.partial(
    pl.kernel,
    out_type=jax.ShapeDtypeStruct((32 * NDEG,), jnp.float32),
    mesh=_vmesh,
    compiler_params=pltpu.CompilerParams(needs_layout_passes=False),
    scratch_types=[
        pltpu.VMEM((NDEG,), jnp.float32),
        pltpu.VMEM((NCH // 32, CHUNK), jnp.int32),
    ],
)
def _deg_kernel(dst_hbm, zdeg_hbm, out_hbm, dloc, didx):
    c = lax.axis_index("c")
    s = lax.axis_index("s")
    wid = c * NTILE + s
    nch = NCH // 32
    pltpu.sync_copy(zdeg_hbm, dloc)
    pltpu.sync_copy(dst_hbm.at[pl.ds(wid * nch, nch)], didx)
    ones = jnp.full((LANES,), 1.0, jnp.float32)

    @pl.loop(0, nch)
    def _(j):
        for k in range(CHUNK // LANES):
            idx = didx[j, pl.ds(k * LANES, LANES)]
            plsc.addupdate_scatter(dloc, [idx], ones)

    pltpu.sync_copy(dloc, out_hbm.at[pl.ds(wid * NDEG, NDEG)])


@functools.partial(
    pl.kernel,
    out_type=jax.ShapeDtypeStruct((2 * NACC, 64), jnp.float32),
    mesh=_vmesh,
    compiler_params=pltpu.CompilerParams(needs_layout_passes=False,
                                         use_tc_tiling_on_sc=False),
    scratch_types=(
        [pltpu.VMEM((G, CHUNK), jnp.int32),
         pltpu.VMEM((G, CHUNK), jnp.int32)]
        + [pltpu.VMEM((CHUNK, 64), jnp.float32) for _ in range(NBUF)]
        + [pltpu.VMEM_SHARED((NACC, 64), jnp.float32),
           pltpu.VMEM_SHARED((NACC, 64), jnp.float32)]
        + [pltpu.SemaphoreType.DMA for _ in range(2 * NBUF)]
    ),
)
def _agg64(xpair_hbm, src_hbm, dst_hbm, zrows_hbm, out_hbm,
           sidx, didx, *rest):
    bufs = rest[:NBUF]
    xsp = rest[NBUF]
    acc = rest[NBUF + 1]
    gsems = rest[NBUF + 2:NBUF + 2 + NBUF]
    ssems = rest[NBUF + 2 + NBUF:]
    c = lax.axis_index("c")
    s = lax.axis_index("s")
    rsl = pl.ds(s * _RZ, _RZ)
    pltpu.sync_copy(xpair_hbm.at[pl.ds(c * NACC + s * _RZ, _RZ)],
                    xsp.at[rsl])
    pltpu.sync_copy(zrows_hbm, acc.at[rsl])
    base = s * _NCH_T
    plsc.subcore_barrier()

    @pl.loop(0, _NCH_T // G)
    def _(g):
        pltpu.sync_copy(src_hbm.at[pl.ds(base + g * G, G)], sidx)
        pltpu.sync_copy(dst_hbm.at[pl.ds(base + g * G, G)], didx)
        slag = NBUF - GAHEAD
        gd = [None] * G
        sd = [None] * G
        for j in range(min(GAHEAD, G)):
            gd[j] = pltpu.async_copy(
                xsp.at[sidx.at[j]], bufs[j % NBUF], gsems[j % NBUF])
        for j in range(G):
            b = j % NBUF
            if j >= slag and sd[j - slag] is not None:
                sd[j - slag].wait()
            gd[j].wait()
            if j + GAHEAD < G:
                nb = (j + GAHEAD) % NBUF
                gd[j + GAHEAD] = pltpu.async_copy(
                    xsp.at[sidx.at[j + GAHEAD]], bufs[nb], gsems[nb])
            sd[j] = pltpu.async_copy(
                bufs[b], acc.at[didx.at[j]], ssems[b], add=True)
        for j in range(max(0, G - slag), G):
            if sd[j] is not None:
                sd[j].wait()

    plsc.subcore_barrier()
    pltpu.sync_copy(acc.at[rsl], out_hbm.at[pl.ds(c * NACC + s * _RZ, _RZ)])


def _dinv_col(degp_ref):
    deg = jnp.sum(degp_ref[...], axis=1, keepdims=True) + 1.0
    return lax.rsqrt(deg)


def _mm(a, b):
    return lax.dot_general(a, b, (((1,), (0,)), ((), ())),
                           preferred_element_type=jnp.float32,
                           precision=lax.Precision.HIGHEST)


def _a_body(degp_ref, x_ref, xp_ref):
    dcol = _dinv_col(degp_ref)
    xsq = dcol * x_ref[...]
    xp_ref[0] = xsq[:, :64]
    xp_ref[1] = xsq[:, 64:]


def _b_body(degp_ref, s1_ref, x_ref, w1_ref, b1_ref,
            h1_ref, hpa_ref, hpb_ref):
    dcol = _dinv_col(degp_ref)
    ssum = jnp.concatenate([s1_ref[0], s1_ref[1]], axis=1)
    z1 = dcol * ssum + (dcol * dcol) * x_ref[...]
    h1 = jnp.maximum(_mm(z1, w1_ref[...]) + b1_ref[...], 0.0)
    h1_ref[...] = h1
    hs = dcol * h1
    hpa_ref[0] = hs[:, 0:64]
    hpa_ref[1] = hs[:, 64:128]
    hpb_ref[0] = hs[:, 128:192]
    hpb_ref[1] = hs[:, 192:256]


def _c_body(degp_ref, s2a_ref, s2b_ref, h1_ref, w2_ref, w3_ref, b2_ref,
            t_ref, tp_ref):
    dcol = _dinv_col(degp_ref)
    ssum = jnp.concatenate(
        [s2a_ref[0], s2a_ref[1], s2b_ref[0], s2b_ref[1]], axis=1)
    z2 = dcol * ssum + (dcol * dcol) * h1_ref[...]
    h2 = jnp.maximum(_mm(z2, w2_ref[...]) + b2_ref[...], 0.0)
    t = _mm(h2, w3_ref[...])
    t_ref[...] = t
    ts = dcol * t
    tp_ref[0] = ts[:, :64]
    tp_ref[1] = ts[:, 64:]


def _d_body(degp_ref, s3_ref, t_ref, b3_ref, o_ref):
    dcol = _dinv_col(degp_ref)
    ssum = jnp.concatenate([s3_ref[0], s3_ref[1]], axis=1)
    o_ref[...] = (dcol * ssum + (dcol * dcol) * t_ref[...] + b3_ref[...])


def _rowspec(w):
    return pl.BlockSpec((RB, w), lambda i: (i, 0))


def _pairspec():
    return pl.BlockSpec((2, RB, 64), lambda i: (0, i, 0))


def _fullspec(shape):
    return pl.BlockSpec(shape, lambda i: tuple(0 for _ in shape))


def _pair_out():
    return jax.ShapeDtypeStruct((2, NACC, 64), jnp.float32)


_stage_a = pl.pallas_call(
    _a_body, grid=(N // RB,),
    in_specs=[_rowspec(32), _rowspec(128)],
    out_specs=_pairspec(),
    out_shape=_pair_out())

_stage_b = pl.pallas_call(
    _b_body, grid=(N // RB,),
    in_specs=[_rowspec(32), _pairspec(), _rowspec(128),
              _fullspec((128, 256)), _fullspec((1, 256))],
    out_specs=[_rowspec(256), _pairspec(), _pairspec()],
    out_shape=[jax.ShapeDtypeStruct((N, 256), jnp.float32),
               _pair_out(), _pair_out()])

_stage_c = pl.pallas_call(
    _c_body, grid=(N // RB,),
    in_specs=[_rowspec(32), _pairspec(), _pairspec(), _rowspec(256),
              _fullspec((256, 256)), _fullspec((256, 128)),
              _fullspec((1, 256))],
    out_specs=[_rowspec(128), _pairspec()],
    out_shape=[jax.ShapeDtypeStruct((N, 128), jnp.float32), _pair_out()])

_stage_d = pl.pallas_call(
    _d_body, grid=(N // RB,),
    in_specs=[_rowspec(32), _pairspec(), _rowspec(128),
              _fullspec((1, 128))],
    out_specs=_rowspec(128),
    out_shape=jax.ShapeDtypeStruct((N, 128), jnp.float32))


def kernel(x, edge_index, W1, b1, W2, b2, W3, b3):
    src = edge_index[0].astype(jnp.int32)
    dst = edge_index[1].astype(jnp.int32)
    pad = EPAD - E
    srcp = jnp.concatenate([src, jnp.zeros((pad,), jnp.int32)]).reshape(NCH, CHUNK)
    dstp = jnp.concatenate([dst, jnp.full((pad,), N, jnp.int32)]).reshape(NCH, CHUNK)
    zdeg = jnp.zeros((NDEG,), jnp.float32)
    zrows = jnp.zeros((_RZ, 64), jnp.float32)

    def agg(pair):
        flat = _agg64(pair.reshape(2 * NACC, 64), srcp, dstp, zrows)
        return flat.reshape(2, NACC, 64)

    degp = _deg_kernel(dstp, zdeg)
    degpt = degp.reshape(32, NDEG).T

    xp = _stage_a(degpt, x)
    s1 = agg(xp)
    h1, hpa, hpb = _stage_b(degpt, s1, x, W1, b1.reshape(1, HID))
    s2a = agg(hpa)
    s2b = agg(hpb)
    t, tp = _stage_c(degpt, s2a, s2b, h1, W2, W3, b2.reshape(1, HID))
    s3 = agg(tp)
    out = _stage_d(degpt, s3, t, b3.reshape(1, F_OUT))
    return out


# speedup vs baseline: 2.3315x; 1.1044x over previous
import functools

import jax
import jax.numpy as jnp
from jax import lax
from jax.experimental import pallas as pl
from jax.experimental.pallas import tpu as pltpu
from jax.experimental.pallas import tpu_sc as plsc

N = 10000
E = 320000
HID = 256
F_OUT = 128

CHUNK = 128
EPAD = 327680
NCH = EPAD // CHUNK
NDEG = 10240
NACC = 10112
NTILE = 16
LANES = 16
NBUF = 5
GAHEAD = 2
G = 16
RB = 2000

_NCH_T = NCH // NTILE
_RZ = NACC // NTILE

_vmesh = plsc.VectorSubcoreMesh(core_axis_name="c", subcore_axis_name="s")


@functools.partial(
    pl.kernel,
    out_type=jax.ShapeDtypeStruct((32 * NDEG,), jnp.float32),
    mesh=_vmesh,
    compiler_params=pltpu.CompilerParams(needs_layout_passes=False),
    scratch_types=[
        pltpu.VMEM((NDEG,), jnp.float32),
        pltpu.VMEM((NCH // 32, CHUNK), jnp.int32),
    ],
)
def _deg_kernel(dst_hbm, zdeg_hbm, out_hbm, dloc, didx):
    c = lax.axis_index("c")
    s = lax.axis_index("s")
    wid = c * NTILE + s
    nch = NCH // 32
    pltpu.sync_copy(zdeg_hbm, dloc)
    pltpu.sync_copy(dst_hbm.at[pl.ds(wid * nch, nch)], didx)
    ones = jnp.full((LANES,), 1.0, jnp.float32)

    @pl.loop(0, nch)
    def _(j):
        for k in range(CHUNK // LANES):
            idx = didx[j, pl.ds(k * LANES, LANES)]
            plsc.addupdate_scatter(dloc, [idx], ones)

    pltpu.sync_copy(dloc, out_hbm.at[pl.ds(wid * NDEG, NDEG)])


@functools.partial(
    pl.kernel,
    out_type=jax.ShapeDtypeStruct((2 * NACC, 64), jnp.float32),
    mesh=_vmesh,
    compiler_params=pltpu.CompilerParams(needs_layout_passes=False,
                                         use_tc_tiling_on_sc=False),
    scratch_types=(
        [pltpu.VMEM((G, CHUNK), jnp.int32),
         pltpu.VMEM((G, CHUNK), jnp.int32)]
        + [pltpu.VMEM((CHUNK, 64), jnp.float32) for _ in range(NBUF)]
        + [pltpu.VMEM_SHARED((NACC, 64), jnp.float32),
           pltpu.VMEM_SHARED((NACC, 64), jnp.float32)]
        + [pltpu.SemaphoreType.DMA for _ in range(2 * NBUF)]
    ),
)
def _agg64(xpair_hbm, src_hbm, dst_hbm, zrows_hbm, out_hbm,
           sidx, didx, *rest):
    bufs = rest[:NBUF]
    xsp = rest[NBUF]
    acc = rest[NBUF + 1]
    gsems = rest[NBUF + 2:NBUF + 2 + NBUF]
    ssems = rest[NBUF + 2 + NBUF:]
    c = lax.axis_index("c")
    s = lax.axis_index("s")
    rsl = pl.ds(s * _RZ, _RZ)
    pltpu.sync_copy(xpair_hbm.at[pl.ds(c * NACC + s * _RZ, _RZ)],
                    xsp.at[rsl])
    pltpu.sync_copy(zrows_hbm, acc.at[rsl])
    base = s * _NCH_T
    plsc.subcore_barrier()

    @pl.loop(0, _NCH_T // G)
    def _(g):
        pltpu.sync_copy(src_hbm.at[pl.ds(base + g * G, G)], sidx)
        pltpu.sync_copy(dst_hbm.at[pl.ds(base + g * G, G)], didx)
        slag = NBUF - GAHEAD
        gd = [None] * G
        sd = [None] * G
        for j in range(min(GAHEAD, G)):
            gd[j] = pltpu.async_copy(
                xsp.at[sidx.at[j]], bufs[j % NBUF], gsems[j % NBUF])
        for j in range(G):
            b = j % NBUF
            if j >= slag and sd[j - slag] is not None:
                sd[j - slag].wait()
            gd[j].wait()
            if j + GAHEAD < G:
                nb = (j + GAHEAD) % NBUF
                gd[j + GAHEAD] = pltpu.async_copy(
                    xsp.at[sidx.at[j + GAHEAD]], bufs[nb], gsems[nb])
            sd[j] = pltpu.async_copy(
                bufs[b], acc.at[didx.at[j]], ssems[b], add=True)
        for j in range(max(0, G - slag), G):
            if sd[j] is not None:
                sd[j].wait()

    plsc.subcore_barrier()
    pltpu.sync_copy(acc.at[rsl], out_hbm.at[pl.ds(c * NACC + s * _RZ, _RZ)])


def _dinv_col(degp_ref):
    deg = jnp.sum(degp_ref[...], axis=1, keepdims=True) + 1.0
    return lax.rsqrt(deg)


def _mm(a, b):
    return lax.dot_general(a, b, (((1,), (0,)), ((), ())),
                           preferred_element_type=jnp.float32,
                           precision=lax.Precision.HIGHEST)


def _a_body(degp_ref, x_ref, xp_ref):
    dcol = _dinv_col(degp_ref)
    xsq = dcol * x_ref[...]
    xp_ref[0] = xsq[:, :64]
    xp_ref[1] = xsq[:, 64:]


def _b_body(degp_ref, s1_ref, x_ref, w1_ref, b1_ref,
            h1_ref, hpa_ref, hpb_ref):
    dcol = _dinv_col(degp_ref)
    ssum = jnp.concatenate([s1_ref[0], s1_ref[1]], axis=1)
    z1 = dcol * ssum + (dcol * dcol) * x_ref[...]
    h1 = jnp.maximum(_mm(z1, w1_ref[...]) + b1_ref[...], 0.0)
    h1_ref[...] = h1
    hs = dcol * h1
    hpa_ref[0] = hs[:, 0:64]
    hpa_ref[1] = hs[:, 64:128]
    hpb_ref[0] = hs[:, 128:192]
    hpb_ref[1] = hs[:, 192:256]


def _c_body(degp_ref, s2a_ref, s2b_ref, h1_ref, w2_ref, w3_ref, b2_ref,
            t_ref, tp_ref):
    dcol = _dinv_col(degp_ref)
    ssum = jnp.concatenate(
        [s2a_ref[0], s2a_ref[1], s2b_ref[0], s2b_ref[1]], axis=1)
    z2 = dcol * ssum + (dcol * dcol) * h1_ref[...]
    h2 = jnp.maximum(_mm(z2, w2_ref[...]) + b2_ref[...], 0.0)
    t = _mm(h2, w3_ref[...])
    t_ref[...] = t
    ts = dcol * t
    tp_ref[0] = ts[:, :64]
    tp_ref[1] = ts[:, 64:]


def _d_body(degp_ref, s3_ref, t_ref, b3_ref, o_ref):
    dcol = _dinv_col(degp_ref)
    ssum = jnp.concatenate([s3_ref[0], s3_ref[1]], axis=1)
    o_ref[...] = (dcol * ssum + (dcol * dcol) * t_ref[...] + b3_ref[...])


def _rowspec(w):
    return pl.BlockSpec((RB, w), lambda i: (i, 0))


def _pairspec():
    return pl.BlockSpec((2, RB, 64), lambda i: (0, i, 0))


def _fullspec(shape):
    return pl.BlockSpec(shape, lambda i: tuple(0 for _ in shape))


def _pair_out():
    return jax.ShapeDtypeStruct((2, NACC, 64), jnp.float32)


_stage_a = pl.pallas_call(
    _a_body, grid=(N // RB,),
    in_specs=[_rowspec(32), _rowspec(128)],
    out_specs=_pairspec(),
    out_shape=_pair_out())

_stage_b = pl.pallas_call(
    _b_body, grid=(N // RB,),
    in_specs=[_rowspec(32), _pairspec(), _rowspec(128),
              _fullspec((128, 256)), _fullspec((1, 256))],
    out_specs=[_rowspec(256), _pairspec(), _pairspec()],
    out_shape=[jax.ShapeDtypeStruct((N, 256), jnp.float32),
               _pair_out(), _pair_out()])

_stage_c = pl.pallas_call(
    _c_body, grid=(N // RB,),
    in_specs=[_rowspec(32), _pairspec(), _pairspec(), _rowspec(256),
              _fullspec((256, 256)), _fullspec((256, 128)),
              _fullspec((1, 256))],
    out_specs=[_rowspec(128), _pairspec()],
    out_shape=[jax.ShapeDtypeStruct((N, 128), jnp.float32), _pair_out()])

_stage_d = pl.pallas_call(
    _d_body, grid=(N // RB,),
    in_specs=[_rowspec(32), _pairspec(), _rowspec(128),
              _fullspec((1, 128))],
    out_specs=_rowspec(128),
    out_shape=jax.ShapeDtypeStruct((N, 128), jnp.float32))


def kernel(x, edge_index, W1, b1, W2, b2, W3, b3):
    src = edge_index[0].astype(jnp.int32)
    dst = edge_index[1].astype(jnp.int32)
    pad = EPAD - E
    srcp = jnp.concatenate([src, jnp.zeros((pad,), jnp.int32)]).reshape(NCH, CHUNK)
    dstp = jnp.concatenate([dst, jnp.full((pad,), N, jnp.int32)]).reshape(NCH, CHUNK)
    zdeg = jnp.zeros((NDEG,), jnp.float32)
    zrows = jnp.zeros((_RZ, 64), jnp.float32)

    def agg(pair):
        flat = _agg64(pair.reshape(2 * NACC, 64), srcp, dstp, zrows)
        return flat.reshape(2, NACC, 64)

    degp = _deg_kernel(dstp, zdeg)
    degpt = degp.reshape(32, NDEG).T

    xp = _stage_a(degpt, x)
    s1 = agg(xp)
    h1, hpa, hpb = _stage_b(degpt, s1, x, W1, b1.reshape(1, HID))
    s2a = agg(hpa)
    s2b = agg(hpb)
    t, tp = _stage_c(degpt, s2a, s2b, h1, W2, W3, b2.reshape(1, HID))
    s3 = agg(tp)
    out = _stage_d(degpt, s3, t, b3.reshape(1, F_OUT))
    return out
